# Initial kernel scaffold; baseline (speedup 1.0000x reference)
import jax
import jax.numpy as jnp
from jax.experimental import pallas as pl


def kernel(inputs, index, w):
    raise NotImplementedError("write your pallas kernel here")


# speedup vs baseline: 9.5244x; 9.5244x over previous
import functools

import jax
import jax.numpy as jnp
from jax.experimental import pallas as pl

_UNITS = 32
_EPS = 1e-07
_NUM_SEG = 256
_N_TOK = 2048
_D_FEAT = 128
_FB = 8


def _laf_kernel(xt_ref, idx_ref, w_ref, out_ref):
    eps = _EPS
    def col(row):
        return jnp.transpose(w_ref[row : row + 1, :])

    p_col = jnp.concatenate([jax.nn.relu(col(r)) for r in (1, 3, 5, 7)], axis=0)
    q_col = jnp.concatenate([jax.nn.relu(col(r)) for r in (0, 2, 4, 6)], axis=0)
    ab_col = jnp.concatenate([col(r) for r in (8, 9, 10, 11)], axis=0)

    idx_col = jnp.transpose(idx_ref[:, :])
    seg_iota = jax.lax.broadcasted_iota(jnp.int32, (_N_TOK, _NUM_SEG), 1)
    oh_t = (idx_col == seg_iota).astype(jnp.float32)

    for f in range(_FB):
        x = xt_ref[f : f + 1, :]
        x = jnp.clip(x, eps, 1.0 - eps)
        lx = jnp.log(x)
        l1 = jnp.log(1.0 - x)
        lx_b = jnp.broadcast_to(lx, (_UNITS, _N_TOK))
        l1_b = jnp.broadcast_to(l1, (_UNITS, _N_TOK))
        l_full = jnp.concatenate([lx_b, l1_b, lx_b, l1_b], axis=0)
        e_t = jnp.exp(p_col * l_full)
        acc = jax.lax.dot(e_t, oh_t, preferred_element_type=jnp.float32)
        s = jnp.maximum(acc, eps)
        sq = jnp.exp(q_col * jnp.log(s))
        terms = sq * ab_col
        num = terms[0:32, :] + terms[32:64, :]
        den = terms[64:96, :] + terms[96:128, :]
        mult = 2.0 * jax.nn.relu(jnp.sign(den)) - 1.0
        den = jnp.where((den < eps) & (den > -eps), mult * eps, den)
        res = num / den
        out_ref[:, f, :] = jnp.transpose(res)


@jax.jit
def kernel(inputs, index, w):
    xt = jnp.transpose(inputs)
    idx2d = index.reshape(1, _N_TOK)
    grid = _D_FEAT // _FB
    out = pl.pallas_call(
        _laf_kernel,
        grid=(grid,),
        in_specs=[
            pl.BlockSpec((_FB, _N_TOK), lambda i: (i, 0)),
            pl.BlockSpec((1, _N_TOK), lambda i: (0, 0)),
            pl.BlockSpec((12, _UNITS), lambda i: (0, 0)),
        ],
        out_specs=pl.BlockSpec((_NUM_SEG, _FB, _UNITS), lambda i: (0, i, 0)),
        out_shape=jax.ShapeDtypeStruct((_NUM_SEG, _D_FEAT, _UNITS), jnp.float32),
    )(xt, idx2d, w)
    return out
